# baseline (speedup 1.0000x reference)
import jax
import jax.numpy as jnp
from jax import lax
from jax.experimental import pallas as pl
from jax.experimental.pallas import tpu as pltpu
from jax.experimental.pallas import tpu_sc as plsc

_B = 4096
_D = 32
_W = 128
_NC, _NS = 2, 16
_NW = _NC * _NS
_BPW = _B // _NW


def _sc_gather_body(uhi, ihi, mfu, mfi, mlu, mli,
                    out_mfu, out_mfi, out_mlu, out_mli,
                    idx_u, idx_i, buf_a, buf_b, buf_c, buf_d, sem):
    wid = lax.axis_index("s") * _NC + lax.axis_index("c")
    base = wid * _BPW
    pltpu.sync_copy(uhi.at[pl.ds(base, _BPW)], idx_u)
    pltpu.sync_copy(ihi.at[pl.ds(base, _BPW)], idx_i)
    c1 = pltpu.async_copy(mfu.at[idx_u], buf_a, sem)
    c2 = pltpu.async_copy(mfi.at[idx_i], buf_b, sem)
    c3 = pltpu.async_copy(mlu.at[idx_u], buf_c, sem)
    c4 = pltpu.async_copy(mli.at[idx_i], buf_d, sem)
    c1.wait()
    pltpu.sync_copy(buf_a, out_mfu.at[pl.ds(base, _BPW)])
    c2.wait()
    pltpu.sync_copy(buf_b, out_mfi.at[pl.ds(base, _BPW)])
    c3.wait()
    pltpu.sync_copy(buf_c, out_mlu.at[pl.ds(base, _BPW)])
    c4.wait()
    pltpu.sync_copy(buf_d, out_mli.at[pl.ds(base, _BPW)])


@jax.jit
def _sc_gather(uhi, ihi, mfu, mfi, mlu, mli):
    mesh = plsc.VectorSubcoreMesh(
        core_axis_name="c", subcore_axis_name="s",
        num_cores=_NC, num_subcores=_NS)
    f32 = jnp.float32
    return pl.kernel(
        _sc_gather_body,
        out_type=[jax.ShapeDtypeStruct((_B, _W), f32)] * 4,
        mesh=mesh,
        scratch_types=[
            pltpu.VMEM((_BPW,), jnp.int32),
            pltpu.VMEM((_BPW,), jnp.int32),
            pltpu.VMEM((_BPW, _W), f32),
            pltpu.VMEM((_BPW, _W), f32),
            pltpu.VMEM((_BPW, _W), f32),
            pltpu.VMEM((_BPW, _W), f32),
            pltpu.SemaphoreType.DMA,
        ],
    )(uhi, ihi, mfu, mfi, mlu, mli)


def _tc_mlp_body(bmfu_ref, bmfi_ref, bmlu_ref, bmli_ref, ulo_ref, ilo_ref,
                 sel_ref, w1u_ref, w1i_ref, b1_ref, w2_ref, b2_ref,
                 w3_ref, b3_ref, wp_mf_ref, wp_mlp_ref, bp_ref, out_ref):
    blk = lax.broadcasted_iota(jnp.int32, (_B, _W), 1) >> 5
    mu = blk == ulo_ref[...]
    mi = blk == ilo_ref[...]
    zero = jnp.zeros((), jnp.float32)
    dot = lambda a, b: jnp.dot(a, b, preferred_element_type=jnp.float32)
    sel = sel_ref[...]
    mf = (dot(jnp.where(mu, bmfu_ref[...], zero), sel)
          * dot(jnp.where(mi, bmfi_ref[...], zero), sel))
    h = jnp.maximum(
        dot(jnp.where(mu, bmlu_ref[...], zero), w1u_ref[...])
        + dot(jnp.where(mi, bmli_ref[...], zero), w1i_ref[...])
        + b1_ref[...][None, :], 0.0)
    h = jnp.maximum(dot(h, w2_ref[...]) + b2_ref[...][None, :], 0.0)
    h = jnp.maximum(dot(h, w3_ref[...]) + b3_ref[...][None, :], 0.0)
    logit = (jnp.sum(mf * wp_mf_ref[...][None, :], axis=1, keepdims=True)
             + jnp.sum(h * wp_mlp_ref[...][None, :], axis=1, keepdims=True)
             + bp_ref[...][None, :])
    out_ref[...] = jax.nn.sigmoid(logit)


@jax.jit
def _tc_mlp(bmfu, bmfi, bmlu, bmli, ulo, ilo,
            sel, w1u, w1i, b1, w2, b2, w3, b3, wp_mf, wp_mlp, bp):
    return pl.pallas_call(
        _tc_mlp_body,
        out_shape=jax.ShapeDtypeStruct((_B, 1), jnp.float32),
    )(bmfu, bmfi, bmlu, bmli, ulo, ilo,
      sel, w1u, w1i, b1, w2, b2, w3, b3, wp_mf, wp_mlp, bp)


def kernel(user_ids, item_ids, mf_user_table, mf_item_table,
           mlp_user_table, mlp_item_table, W1, b1, W2, b2, W3, b3, Wp, bp):
    uids = user_ids.astype(jnp.int32)
    iids = item_ids.astype(jnp.int32)
    uhi = uids >> 2
    ihi = iids >> 2
    ulo = (uids & 3).reshape(_B, 1)
    ilo = (iids & 3).reshape(_B, 1)
    bmfu, bmfi, bmlu, bmli = _sc_gather(
        uhi, ihi,
        mf_user_table.reshape(-1, _W), mf_item_table.reshape(-1, _W),
        mlp_user_table.reshape(-1, _W), mlp_item_table.reshape(-1, _W))
    sel = jnp.tile(jnp.eye(_D, dtype=jnp.float32), (_W // _D, 1))
    w1u = jnp.tile(W1[:_D, :], (_W // _D, 1))
    w1i = jnp.tile(W1[_D:, :], (_W // _D, 1))
    return _tc_mlp(
        bmfu, bmfi, bmlu, bmli, ulo, ilo,
        sel, w1u, w1i, b1, W2, b2, W3, b3,
        Wp[:_D, 0], Wp[_D:, 0], bp)
